# baseline (speedup 1.0000x reference)
import jax
import jax.numpy as jnp
from jax import lax
from jax.experimental import pallas as pl
from jax.experimental.pallas import tpu as pltpu
from jax.experimental.pallas import tpu_sc as plsc

DIM = 64
NWORKERS = 32
W = 800


def kernel(x, table):
    B, S = x.shape
    n = B * S
    idx = x.reshape(n)
    b_per_w = n // NWORKERS
    steps = b_per_w // W

    mesh = plsc.VectorSubcoreMesh(core_axis_name="c", subcore_axis_name="s")
    cp = pltpu.CompilerParams(use_tc_tiling_on_sc=False)

    @pl.kernel(
        out_type=jax.ShapeDtypeStruct((B, S, DIM), table.dtype),
        mesh=mesh,
        scratch_types=[
            pltpu.VMEM((W,), jnp.int32),
            pltpu.VMEM((W,), jnp.int32),
            pltpu.VMEM((W, DIM), jnp.float32),
            pltpu.VMEM((W, DIM), jnp.float32),
            pltpu.SemaphoreType.DMA,
            pltpu.SemaphoreType.DMA,
            pltpu.SemaphoreType.DMA,
            pltpu.SemaphoreType.DMA,
        ],
        compiler_params=cp,
    )
    def gather_kernel(
        table_hbm, i_hbm, o_hbm, idx0, idx1, g0, g1, si0, si1, sg0, sg1
    ):
        wid = lax.axis_index("s") * 2 + lax.axis_index("c")
        base_w = wid * b_per_w
        o2d = o_hbm
        idxb, gb, sib, sgb = [idx0, idx1], [g0, g1], [si0, si1], [sg0, sg1]

        ih = [None, None]
        gh = [None, None]
        ih[0] = pltpu.async_copy(i_hbm.at[pl.ds(base_w, W)], idxb[0], sib[0])
        for c in range(steps):
            cur = c & 1
            prv = cur ^ 1
            ih[cur].wait()
            gh[cur] = pltpu.async_copy(
                table_hbm.at[idxb[cur]], gb[cur], sgb[cur]
            )
            if c > 0:
                gh[prv].wait()
                b0 = (base_w + (c - 1) * W) // S
                for k in range(W // S):
                    pltpu.sync_copy(
                        gb[prv].at[pl.ds(k * S, S)], o2d.at[b0 + k]
                    )
            if c + 1 < steps:
                ih[prv] = pltpu.async_copy(
                    i_hbm.at[pl.ds(base_w + (c + 1) * W, W)], idxb[prv], sib[prv]
                )
        last = (steps - 1) & 1
        gh[last].wait()
        b0 = (base_w + (steps - 1) * W) // S
        for k in range(W // S):
            pltpu.sync_copy(gb[last].at[pl.ds(k * S, S)], o2d.at[b0 + k])

    return gather_kernel(table, idx)


# speedup vs baseline: 1.0044x; 1.0044x over previous
import jax
import jax.numpy as jnp
from jax import lax
from jax.experimental import pallas as pl
from jax.experimental.pallas import tpu as pltpu
from jax.experimental.pallas import tpu_sc as plsc

DIM = 64
NWORKERS = 32
W = 800


def kernel(x, table):
    B, S = x.shape
    n = B * S
    idx = x.reshape(n)
    b_per_w = n // NWORKERS
    steps = b_per_w // W

    mesh = plsc.VectorSubcoreMesh(core_axis_name="c", subcore_axis_name="s")
    cp = pltpu.CompilerParams(use_tc_tiling_on_sc=False)

    @pl.kernel(
        out_type=jax.ShapeDtypeStruct((n, DIM), table.dtype),
        mesh=mesh,
        scratch_types=[
            pltpu.VMEM((W,), jnp.int32),
            pltpu.VMEM((W,), jnp.int32),
            pltpu.VMEM((W, DIM), jnp.float32),
            pltpu.VMEM((W, DIM), jnp.float32),
            pltpu.SemaphoreType.DMA,
            pltpu.SemaphoreType.DMA,
            pltpu.SemaphoreType.DMA,
            pltpu.SemaphoreType.DMA,
        ],
        compiler_params=cp,
    )
    def gather_kernel(
        table_hbm, i_hbm, o_hbm, idx0, idx1, g0, g1, si0, si1, sg0, sg1
    ):
        wid = lax.axis_index("s") * 2 + lax.axis_index("c")
        base_w = wid * b_per_w
        o2d = o_hbm
        idxb, gb, sib, sgb = [idx0, idx1], [g0, g1], [si0, si1], [sg0, sg1]

        ih = [None, None]
        gh = [None, None]
        ih[0] = pltpu.async_copy(i_hbm.at[pl.ds(base_w, W)], idxb[0], sib[0])
        for c in range(steps):
            cur = c & 1
            prv = cur ^ 1
            ih[cur].wait()
            gh[cur] = pltpu.async_copy(
                table_hbm.at[idxb[cur]], gb[cur], sgb[cur]
            )
            if c > 0:
                gh[prv].wait()
                pltpu.sync_copy(
                    gb[prv], o2d.at[pl.ds(base_w + (c - 1) * W, W)]
                )
            if c + 1 < steps:
                ih[prv] = pltpu.async_copy(
                    i_hbm.at[pl.ds(base_w + (c + 1) * W, W)], idxb[prv], sib[prv]
                )
        last = (steps - 1) & 1
        gh[last].wait()
        pltpu.sync_copy(gb[last], o2d.at[pl.ds(base_w + (steps - 1) * W, W)])

    return gather_kernel(table, idx).reshape(B, S, DIM)


# speedup vs baseline: 1.0063x; 1.0019x over previous
import jax
import jax.numpy as jnp
from jax import lax
from jax.experimental import pallas as pl
from jax.experimental.pallas import tpu as pltpu
from jax.experimental.pallas import tpu_sc as plsc

DIM = 64
NWORKERS = 32
W = 800


def kernel(x, table):
    B, S = x.shape
    n = B * S
    idx = jnp.maximum(x.reshape(n), 0)
    b_per_w = n // NWORKERS
    steps = b_per_w // W

    mesh = plsc.VectorSubcoreMesh(core_axis_name="c", subcore_axis_name="s")
    cp = pltpu.CompilerParams(use_tc_tiling_on_sc=False)

    @pl.kernel(
        out_type=jax.ShapeDtypeStruct((n, DIM), table.dtype),
        mesh=mesh,
        scratch_types=[
            pltpu.VMEM((W,), jnp.int32),
            pltpu.VMEM((W,), jnp.int32),
            pltpu.VMEM((W, DIM), jnp.float32),
            pltpu.VMEM((W, DIM), jnp.float32),
            pltpu.SemaphoreType.DMA,
            pltpu.SemaphoreType.DMA,
            pltpu.SemaphoreType.DMA,
            pltpu.SemaphoreType.DMA,
        ],
        compiler_params=cp,
    )
    def gather_kernel(
        table_hbm, i_hbm, o_hbm, idx0, idx1, g0, g1, si0, si1, sg0, sg1
    ):
        wid = lax.axis_index("s") * 2 + lax.axis_index("c")
        base_w = wid * b_per_w
        o2d = o_hbm
        idxb, gb, sib, sgb = [idx0, idx1], [g0, g1], [si0, si1], [sg0, sg1]

        ih = [None, None]
        gh = [None, None]
        ih[0] = pltpu.async_copy(i_hbm.at[pl.ds(base_w, W)], idxb[0], sib[0])
        for c in range(steps):
            cur = c & 1
            prv = cur ^ 1
            ih[cur].wait()
            gh[cur] = pltpu.async_copy(
                table_hbm.at[idxb[cur]], gb[cur], sgb[cur]
            )
            if c > 0:
                gh[prv].wait()
                pltpu.sync_copy(
                    gb[prv], o2d.at[pl.ds(base_w + (c - 1) * W, W)]
                )
            if c + 1 < steps:
                ih[prv] = pltpu.async_copy(
                    i_hbm.at[pl.ds(base_w + (c + 1) * W, W)], idxb[prv], sib[prv]
                )
        last = (steps - 1) & 1
        gh[last].wait()
        pltpu.sync_copy(gb[last], o2d.at[pl.ds(base_w + (steps - 1) * W, W)])

    return gather_kernel(table, idx).reshape(B, S, DIM)
